# baseline (speedup 1.0000x reference)
import functools

import jax
import jax.numpy as jnp
from jax import lax
from jax.experimental import pallas as pl
from jax.experimental.pallas import tpu as pltpu
from jax.experimental.pallas import tpu_sc as plsc

N = 10000
E = 320000
D_IN = 128
H = 64
L = 4
MASK_RATIO = 0.15
EPS_BN = 1e-5

NC = 2
NS = 16
NW = NC * NS
CH = 128
LN = 128
LP = 88
NR = 2 * LP
CAP = LN * LP
RW = 313
ROWS_PAD = 10112
JROW0 = 10016
NBUF = 4

BLK = 1000
NBLK = N // BLK


def _sc_agg_body(h_hbm, srcw_hbm, dstw_hbm, out_hbm,
                 src_v, dst_v, gb0, gb1, gb2, gb3,
                 agg_sh, sem0, sem1, sem2, sem3):
    c = lax.axis_index("c")
    s = lax.axis_index("s")
    w = c * NS + s

    gbufs = (gb0, gb1, gb2, gb3)
    sems = (sem0, sem1, sem2, sem3)

    pltpu.sync_copy(srcw_hbm.at[w], src_v)
    pltpu.sync_copy(dstw_hbm.at[w], dst_v)

    def _zrow(i, carry):
        for k in range(H // 16):
            gb0[i, pl.ds(16 * k, 16)] = jnp.zeros((16,), jnp.float32)
        return carry
    lax.fori_loop(0, LN, _zrow, 0)
    rbase = w * RW
    off = 0
    for sz in (LN, LN, RW - 2 * LN):
        pltpu.sync_copy(gb0.at[pl.ds(0, sz)], agg_sh.at[pl.ds(rbase + off, sz)])
        off += sz

    def _start(j, b):
        pltpu.make_async_copy(h_hbm.at[src_v.at[j]], gbufs[b], sems[b]).start()

    for b in range(NBUF - 1):
        _start(b, b)

    def _step(t, carry):
        for b in range(NBUF):
            j = t * NBUF + b
            pltpu.make_async_copy(h_hbm.at[src_v.at[j]], gbufs[b], sems[b]).wait()
            pltpu.sync_copy(gbufs[b], agg_sh.at[dst_v.at[j]], add=True)
            nj = j + NBUF - 1

            @pl.when(nj < NR)
            def _():
                _start(nj, (b + NBUF - 1) % NBUF)
        return carry
    lax.fori_loop(0, NR // NBUF, _step, 0)

    pltpu.sync_copy(agg_sh.at[pl.ds(rbase, RW)], out_hbm.at[pl.ds(rbase, RW)])


_sc_agg = functools.partial(
    pl.kernel,
    out_type=jax.ShapeDtypeStruct((ROWS_PAD, H), jnp.float32),
    mesh=plsc.VectorSubcoreMesh(core_axis_name="c", subcore_axis_name="s",
                                num_cores=NC, num_subcores=NS),
    scratch_types=[
        pltpu.VMEM((NR, LN), jnp.int32),
        pltpu.VMEM((NR, LN), jnp.int32),
        pltpu.VMEM((LN, H), jnp.float32),
        pltpu.VMEM((LN, H), jnp.float32),
        pltpu.VMEM((LN, H), jnp.float32),
        pltpu.VMEM((LN, H), jnp.float32),
        pltpu.VMEM_SHARED((ROWS_PAD, H), jnp.float32),
        pltpu.SemaphoreType.DMA,
        pltpu.SemaphoreType.DMA,
        pltpu.SemaphoreType.DMA,
        pltpu.SemaphoreType.DMA,
    ],
    compiler_params=pltpu.CompilerParams(use_tc_tiling_on_sc=False),
)(_sc_agg_body)


def _embed_body(x_ref, m_ref, tok_ref, w_ref, b_ref, h_ref):
    m = m_ref[...]
    xm = x_ref[...] * (1.0 - m) + tok_ref[...] * m
    h_ref[...] = (jnp.dot(xm, w_ref[...], preferred_element_type=jnp.float32)
                  + b_ref[...])


def _embed(x, maskf, tok, w, b):
    return pl.pallas_call(
        _embed_body,
        grid=(NBLK,),
        in_specs=[
            pl.BlockSpec((BLK, D_IN), lambda i: (i, 0)),
            pl.BlockSpec((BLK, 1), lambda i: (i, 0)),
            pl.BlockSpec((1, D_IN), lambda i: (0, 0)),
            pl.BlockSpec((D_IN, H), lambda i: (0, 0)),
            pl.BlockSpec((1, H), lambda i: (0, 0)),
        ],
        out_specs=pl.BlockSpec((BLK, H), lambda i: (i, 0)),
        out_shape=jax.ShapeDtypeStruct((N, H), jnp.float32),
    )(x, maskf, tok, w, b)


def _layer_a_body(h_ref, agg_ref, w1_ref, b1_ref, w2_ref, b2_ref,
                  y_ref, s8_ref, acc_s):
    i = pl.program_id(0)
    z = h_ref[...] + agg_ref[...]
    u = jnp.maximum(
        jnp.dot(z, w1_ref[...], preferred_element_type=jnp.float32)
        + b1_ref[...], 0.0)
    y = (jnp.dot(u, w2_ref[...], preferred_element_type=jnp.float32)
         + b2_ref[...])
    y_ref[...] = y

    @pl.when(i == 0)
    def _():
        acc_s[...] = jnp.zeros_like(acc_s)

    acc_s[...] += jnp.sum(y.reshape(BLK // 8, 8, H), axis=0)

    @pl.when(i == NBLK - 1)
    def _():
        s8_ref[...] = acc_s[...]


def _layer_a(h, agg, w1, b1, w2, b2):
    return pl.pallas_call(
        _layer_a_body,
        grid=(NBLK,),
        in_specs=[
            pl.BlockSpec((BLK, H), lambda i: (i, 0)),
            pl.BlockSpec((BLK, H), lambda i: (i, 0)),
            pl.BlockSpec((H, 2 * H), lambda i: (0, 0)),
            pl.BlockSpec((1, 2 * H), lambda i: (0, 0)),
            pl.BlockSpec((2 * H, H), lambda i: (0, 0)),
            pl.BlockSpec((1, H), lambda i: (0, 0)),
        ],
        out_specs=[
            pl.BlockSpec((BLK, H), lambda i: (i, 0)),
            pl.BlockSpec((8, H), lambda i: (0, 0)),
        ],
        out_shape=[
            jax.ShapeDtypeStruct((N, H), jnp.float32),
            jax.ShapeDtypeStruct((8, H), jnp.float32),
        ],
        scratch_shapes=[
            pltpu.VMEM((8, H), jnp.float32),
        ],
    )(h, agg, w1, b1, w2, b2)


def _layer_s_body(y_ref, s8_ref, st_ref, acc_q, mu_s):
    i = pl.program_id(0)

    @pl.when(i == 0)
    def _():
        mu_s[...] = jnp.sum(s8_ref[...], axis=0, keepdims=True) / N
        acc_q[...] = jnp.zeros_like(acc_q)

    d = y_ref[...] - mu_s[...]
    d = d * d
    acc_q[...] += jnp.sum(d.reshape(BLK // 8, 8, H), axis=0)

    @pl.when(i == NBLK - 1)
    def _():
        st_ref[...] = jnp.concatenate(
            [mu_s[...], jnp.sum(acc_q[...], axis=0, keepdims=True) / N],
            axis=0)


def _layer_s(y, s8):
    return pl.pallas_call(
        _layer_s_body,
        grid=(NBLK,),
        in_specs=[
            pl.BlockSpec((BLK, H), lambda i: (i, 0)),
            pl.BlockSpec((8, H), lambda i: (0, 0)),
        ],
        out_specs=pl.BlockSpec((2, H), lambda i: (0, 0)),
        out_shape=jax.ShapeDtypeStruct((2, H), jnp.float32),
        scratch_shapes=[
            pltpu.VMEM((8, H), jnp.float32),
            pltpu.VMEM((1, H), jnp.float32),
        ],
    )(y, s8)


def _layer_b_body(y_ref, st_ref, bw_ref, bb_ref, o_ref):
    st = st_ref[...]
    mu = st[0:1, :]
    var = st[1:2, :]
    o_ref[...] = jnp.maximum(
        (y_ref[...] - mu) / jnp.sqrt(var + EPS_BN) * bw_ref[...] + bb_ref[...],
        0.0)


def _layer_b(y, st, bw, bb):
    return pl.pallas_call(
        _layer_b_body,
        grid=(NBLK,),
        in_specs=[
            pl.BlockSpec((BLK, H), lambda i: (i, 0)),
            pl.BlockSpec((2, H), lambda i: (0, 0)),
            pl.BlockSpec((1, H), lambda i: (0, 0)),
            pl.BlockSpec((1, H), lambda i: (0, 0)),
        ],
        out_specs=pl.BlockSpec((BLK, H), lambda i: (i, 0)),
        out_shape=jax.ShapeDtypeStruct((N, H), jnp.float32),
    )(y, st, bw, bb)


def _dec_body(h_ref, x_ref, m_ref, w1_ref, b1_ref, w2_ref, b2_ref,
              recon_ref, loss_ref, cnt_ref, acc_l, acc_c):
    i = pl.program_id(0)
    r1 = jnp.maximum(
        jnp.dot(h_ref[...], w1_ref[...], preferred_element_type=jnp.float32)
        + b1_ref[...], 0.0)
    rec = (jnp.dot(r1, w2_ref[...], preferred_element_type=jnp.float32)
           + b2_ref[...])
    recon_ref[...] = rec
    x = x_ref[...]
    pn = jnp.maximum(jnp.sqrt(jnp.sum(rec * rec, axis=1, keepdims=True)), 1e-12)
    xn = jnp.maximum(jnp.sqrt(jnp.sum(x * x, axis=1, keepdims=True)), 1e-12)
    cos = jnp.sum((rec / pn) * (x / xn), axis=1, keepdims=True)
    e = 1.0 - cos
    m = m_ref[...]

    @pl.when(i == 0)
    def _():
        acc_l[...] = jnp.zeros_like(acc_l)
        acc_c[...] = jnp.zeros_like(acc_c)

    acc_l[...] += jnp.sum(e * e * m).reshape(1, 1)
    acc_c[...] += jnp.sum(m).reshape(1, 1)

    @pl.when(i == NBLK - 1)
    def _():
        loss_ref[...] = acc_l[...] / acc_c[...]
        cnt_ref[...] = acc_c[...].astype(jnp.int32)


def _dec(h, x, maskf, w1, b1, w2, b2):
    return pl.pallas_call(
        _dec_body,
        grid=(NBLK,),
        in_specs=[
            pl.BlockSpec((BLK, H), lambda i: (i, 0)),
            pl.BlockSpec((BLK, D_IN), lambda i: (i, 0)),
            pl.BlockSpec((BLK, 1), lambda i: (i, 0)),
            pl.BlockSpec((H, H), lambda i: (0, 0)),
            pl.BlockSpec((1, H), lambda i: (0, 0)),
            pl.BlockSpec((H, D_IN), lambda i: (0, 0)),
            pl.BlockSpec((1, D_IN), lambda i: (0, 0)),
        ],
        out_specs=[
            pl.BlockSpec((BLK, D_IN), lambda i: (i, 0)),
            pl.BlockSpec((1, 1), lambda i: (0, 0)),
            pl.BlockSpec((1, 1), lambda i: (0, 0)),
        ],
        out_shape=[
            jax.ShapeDtypeStruct((N, D_IN), jnp.float32),
            jax.ShapeDtypeStruct((1, 1), jnp.float32),
            jax.ShapeDtypeStruct((1, 1), jnp.int32),
        ],
        scratch_shapes=[
            pltpu.VMEM((1, 1), jnp.float32),
            pltpu.VMEM((1, 1), jnp.float32),
        ],
    )(h, x, maskf, w1, b1, w2, b2)


def kernel(x, edge_index, edge_attr, mask_token, emb_w, emb_b,
           conv_w1, conv_b1, conv_w2, conv_b2, bn_w, bn_b,
           dec_w1, dec_b1, dec_w2, dec_b2):
    mask = jax.random.uniform(jax.random.key(42), (N,)) < MASK_RATIO
    maskf = mask.astype(jnp.float32)[:, None]

    src = edge_index[0]
    dst = edge_index[1]
    order = jnp.argsort(dst, stable=True)
    srcs = src[order]
    dsts = dst[order]
    starts = jnp.searchsorted(
        dsts, (jnp.arange(NW) * RW).astype(jnp.int32)).astype(jnp.int32)
    ends = jnp.concatenate([starts[1:], jnp.array([E], jnp.int32)])
    loc = jnp.arange(CAP, dtype=jnp.int32)[None, :]
    pos = starts[:, None] + loc
    valid = pos < ends[:, None]
    posc = jnp.minimum(pos, E - 1)
    e_src = srcs[posc]
    e_dst = dsts[posc]
    row_start = jnp.searchsorted(dsts, e_dst.reshape(-1)).reshape(NW, CAP)
    lane = loc // LP
    deferred = ((row_start - starts[:, None]) // LP) < lane
    jsrc = (loc * 97) % N
    jdst = (JROW0 + jnp.arange(NW, dtype=jnp.int32))[:, None]
    m1 = valid & ~deferred
    m2 = valid & deferred

    def _grid(sv, dv, m):
        sg = jnp.where(m, sv, jsrc).reshape(NW, LN, LP).transpose(0, 2, 1)
        dg = jnp.where(m, dv, jdst).reshape(NW, LN, LP).transpose(0, 2, 1)
        return sg, dg
    s1, d1 = _grid(e_src, e_dst, m1)
    s2, d2 = _grid(e_src, e_dst, m2)
    srcw = jnp.concatenate([s1, s2], axis=1)
    dstw = jnp.concatenate([d1, d2], axis=1)

    h = _embed(x, maskf, mask_token.reshape(1, D_IN), emb_w,
               emb_b.reshape(1, H))
    for i in range(L):
        agg = _sc_agg(h, srcw, dstw)
        y, s8 = _layer_a(h, agg, conv_w1[i], conv_b1[i].reshape(1, 2 * H),
                         conv_w2[i], conv_b2[i].reshape(1, H))
        st = _layer_s(y, s8)
        h = _layer_b(y, st, bn_w[i].reshape(1, H), bn_b[i].reshape(1, H))

    recon, loss11, cnt11 = _dec(h, x, maskf, dec_w1, dec_b1.reshape(1, H),
                                dec_w2, dec_b2.reshape(1, D_IN))
    return (loss11.reshape(()), recon, mask, x, cnt11.reshape(()))


# speedup vs baseline: 4.0085x; 4.0085x over previous
import functools

import jax
import jax.numpy as jnp
from jax import lax
from jax.experimental import pallas as pl
from jax.experimental.pallas import tpu as pltpu
from jax.experimental.pallas import tpu_sc as plsc

N = 10000
E = 320000
D_IN = 128
H = 64
L = 4
MASK_RATIO = 0.15
EPS_BN = 1e-5

NC = 2
NS = 16
NW = NC * NS
CH = 128
LN = 128
LP = 88
NR = 2 * LP
CAP = LN * LP
RW = 313
ROWS_PAD = 10112
JROW0 = 10016
NBUF = 4

BLK = 1000
NBLK = N // BLK


def _sc_agg_body(h_hbm, srcw_hbm, dstw_hbm, out_hbm,
                 src_v, dst_v, gb0, gb1, gb2, gb3,
                 agg_sh, sem0, sem1, sem2, sem3):
    c = lax.axis_index("c")
    s = lax.axis_index("s")
    w = c * NS + s

    gbufs = (gb0, gb1, gb2, gb3)
    sems = (sem0, sem1, sem2, sem3)

    pltpu.sync_copy(srcw_hbm.at[w], src_v)
    pltpu.sync_copy(dstw_hbm.at[w], dst_v)

    def _zrow(i, carry):
        for k in range(H // 16):
            gb0[i, pl.ds(16 * k, 16)] = jnp.zeros((16,), jnp.float32)
        return carry
    lax.fori_loop(0, LN, _zrow, 0)
    rbase = w * RW
    off = 0
    for sz in (LN, LN, RW - 2 * LN):
        pltpu.sync_copy(gb0.at[pl.ds(0, sz)], agg_sh.at[pl.ds(rbase + off, sz)])
        off += sz

    def _start(j, b):
        pltpu.make_async_copy(h_hbm.at[src_v.at[j]], gbufs[b], sems[b]).start()

    for b in range(NBUF - 1):
        _start(b, b)

    def _step(t, carry):
        for b in range(NBUF):
            j = t * NBUF + b
            pltpu.make_async_copy(h_hbm.at[src_v.at[j]], gbufs[b], sems[b]).wait()
            pltpu.sync_copy(gbufs[b], agg_sh.at[dst_v.at[j]], add=True)
            nj = j + NBUF - 1

            @pl.when(nj < NR)
            def _():
                _start(nj, (b + NBUF - 1) % NBUF)
        return carry
    lax.fori_loop(0, NR // NBUF, _step, 0)

    pltpu.sync_copy(agg_sh.at[pl.ds(rbase, RW)], out_hbm.at[pl.ds(rbase, RW)])


_sc_agg = functools.partial(
    pl.kernel,
    out_type=jax.ShapeDtypeStruct((ROWS_PAD, H), jnp.float32),
    mesh=plsc.VectorSubcoreMesh(core_axis_name="c", subcore_axis_name="s",
                                num_cores=NC, num_subcores=NS),
    scratch_types=[
        pltpu.VMEM((NR, LN), jnp.int32),
        pltpu.VMEM((NR, LN), jnp.int32),
        pltpu.VMEM((LN, H), jnp.float32),
        pltpu.VMEM((LN, H), jnp.float32),
        pltpu.VMEM((LN, H), jnp.float32),
        pltpu.VMEM((LN, H), jnp.float32),
        pltpu.VMEM_SHARED((ROWS_PAD, H), jnp.float32),
        pltpu.SemaphoreType.DMA,
        pltpu.SemaphoreType.DMA,
        pltpu.SemaphoreType.DMA,
        pltpu.SemaphoreType.DMA,
    ],
    compiler_params=pltpu.CompilerParams(use_tc_tiling_on_sc=False),
)(_sc_agg_body)


def _embed_body(x_ref, m_ref, tok_ref, w_ref, b_ref, h_ref):
    m = m_ref[...]
    xm = x_ref[...] * (1.0 - m) + tok_ref[...] * m
    h_ref[...] = (jnp.dot(xm, w_ref[...], preferred_element_type=jnp.float32)
                  + b_ref[...])


def _embed(x, maskf, tok, w, b):
    return pl.pallas_call(
        _embed_body,
        grid=(NBLK,),
        in_specs=[
            pl.BlockSpec((BLK, D_IN), lambda i: (i, 0)),
            pl.BlockSpec((BLK, 1), lambda i: (i, 0)),
            pl.BlockSpec((1, D_IN), lambda i: (0, 0)),
            pl.BlockSpec((D_IN, H), lambda i: (0, 0)),
            pl.BlockSpec((1, H), lambda i: (0, 0)),
        ],
        out_specs=pl.BlockSpec((BLK, H), lambda i: (i, 0)),
        out_shape=jax.ShapeDtypeStruct((N, H), jnp.float32),
    )(x, maskf, tok, w, b)


def _layer_a_body(h_ref, agg_ref, w1_ref, b1_ref, w2_ref, b2_ref,
                  y_ref, s8_ref, acc_s):
    i = pl.program_id(0)
    z = h_ref[...] + agg_ref[...]
    u = jnp.maximum(
        jnp.dot(z, w1_ref[...], preferred_element_type=jnp.float32)
        + b1_ref[...], 0.0)
    y = (jnp.dot(u, w2_ref[...], preferred_element_type=jnp.float32)
         + b2_ref[...])
    y_ref[...] = y

    @pl.when(i == 0)
    def _():
        acc_s[...] = jnp.zeros_like(acc_s)

    acc_s[...] += jnp.sum(y.reshape(BLK // 8, 8, H), axis=0)

    @pl.when(i == NBLK - 1)
    def _():
        s8_ref[...] = acc_s[...]


def _layer_a(h, agg, w1, b1, w2, b2):
    return pl.pallas_call(
        _layer_a_body,
        grid=(NBLK,),
        in_specs=[
            pl.BlockSpec((BLK, H), lambda i: (i, 0)),
            pl.BlockSpec((BLK, H), lambda i: (i, 0)),
            pl.BlockSpec((H, 2 * H), lambda i: (0, 0)),
            pl.BlockSpec((1, 2 * H), lambda i: (0, 0)),
            pl.BlockSpec((2 * H, H), lambda i: (0, 0)),
            pl.BlockSpec((1, H), lambda i: (0, 0)),
        ],
        out_specs=[
            pl.BlockSpec((BLK, H), lambda i: (i, 0)),
            pl.BlockSpec((8, H), lambda i: (0, 0)),
        ],
        out_shape=[
            jax.ShapeDtypeStruct((N, H), jnp.float32),
            jax.ShapeDtypeStruct((8, H), jnp.float32),
        ],
        scratch_shapes=[
            pltpu.VMEM((8, H), jnp.float32),
        ],
    )(h, agg, w1, b1, w2, b2)


def _layer_s_body(y_ref, s8_ref, st_ref, acc_q, mu_s):
    i = pl.program_id(0)

    @pl.when(i == 0)
    def _():
        mu_s[...] = jnp.sum(s8_ref[...], axis=0, keepdims=True) / N
        acc_q[...] = jnp.zeros_like(acc_q)

    d = y_ref[...] - mu_s[...]
    d = d * d
    acc_q[...] += jnp.sum(d.reshape(BLK // 8, 8, H), axis=0)

    @pl.when(i == NBLK - 1)
    def _():
        st_ref[...] = jnp.concatenate(
            [mu_s[...], jnp.sum(acc_q[...], axis=0, keepdims=True) / N],
            axis=0)


def _layer_s(y, s8):
    return pl.pallas_call(
        _layer_s_body,
        grid=(NBLK,),
        in_specs=[
            pl.BlockSpec((BLK, H), lambda i: (i, 0)),
            pl.BlockSpec((8, H), lambda i: (0, 0)),
        ],
        out_specs=pl.BlockSpec((2, H), lambda i: (0, 0)),
        out_shape=jax.ShapeDtypeStruct((2, H), jnp.float32),
        scratch_shapes=[
            pltpu.VMEM((8, H), jnp.float32),
            pltpu.VMEM((1, H), jnp.float32),
        ],
    )(y, s8)


def _layer_b_body(y_ref, st_ref, bw_ref, bb_ref, o_ref):
    st = st_ref[...]
    mu = st[0:1, :]
    var = st[1:2, :]
    o_ref[...] = jnp.maximum(
        (y_ref[...] - mu) / jnp.sqrt(var + EPS_BN) * bw_ref[...] + bb_ref[...],
        0.0)


def _layer_b(y, st, bw, bb):
    return pl.pallas_call(
        _layer_b_body,
        grid=(NBLK,),
        in_specs=[
            pl.BlockSpec((BLK, H), lambda i: (i, 0)),
            pl.BlockSpec((2, H), lambda i: (0, 0)),
            pl.BlockSpec((1, H), lambda i: (0, 0)),
            pl.BlockSpec((1, H), lambda i: (0, 0)),
        ],
        out_specs=pl.BlockSpec((BLK, H), lambda i: (i, 0)),
        out_shape=jax.ShapeDtypeStruct((N, H), jnp.float32),
    )(y, st, bw, bb)


def _dec_body(h_ref, x_ref, m_ref, w1_ref, b1_ref, w2_ref, b2_ref,
              recon_ref, loss_ref, cnt_ref, acc_l, acc_c):
    i = pl.program_id(0)
    r1 = jnp.maximum(
        jnp.dot(h_ref[...], w1_ref[...], preferred_element_type=jnp.float32)
        + b1_ref[...], 0.0)
    rec = (jnp.dot(r1, w2_ref[...], preferred_element_type=jnp.float32)
           + b2_ref[...])
    recon_ref[...] = rec
    x = x_ref[...]
    pn = jnp.maximum(jnp.sqrt(jnp.sum(rec * rec, axis=1, keepdims=True)), 1e-12)
    xn = jnp.maximum(jnp.sqrt(jnp.sum(x * x, axis=1, keepdims=True)), 1e-12)
    cos = jnp.sum((rec / pn) * (x / xn), axis=1, keepdims=True)
    e = 1.0 - cos
    m = m_ref[...]

    @pl.when(i == 0)
    def _():
        acc_l[...] = jnp.zeros_like(acc_l)
        acc_c[...] = jnp.zeros_like(acc_c)

    acc_l[...] += jnp.sum(e * e * m).reshape(1, 1)
    acc_c[...] += jnp.sum(m).reshape(1, 1)

    @pl.when(i == NBLK - 1)
    def _():
        loss_ref[...] = acc_l[...] / acc_c[...]
        cnt_ref[...] = acc_c[...].astype(jnp.int32)


def _dec(h, x, maskf, w1, b1, w2, b2):
    return pl.pallas_call(
        _dec_body,
        grid=(NBLK,),
        in_specs=[
            pl.BlockSpec((BLK, H), lambda i: (i, 0)),
            pl.BlockSpec((BLK, D_IN), lambda i: (i, 0)),
            pl.BlockSpec((BLK, 1), lambda i: (i, 0)),
            pl.BlockSpec((H, H), lambda i: (0, 0)),
            pl.BlockSpec((1, H), lambda i: (0, 0)),
            pl.BlockSpec((H, D_IN), lambda i: (0, 0)),
            pl.BlockSpec((1, D_IN), lambda i: (0, 0)),
        ],
        out_specs=[
            pl.BlockSpec((BLK, D_IN), lambda i: (i, 0)),
            pl.BlockSpec((1, 1), lambda i: (0, 0)),
            pl.BlockSpec((1, 1), lambda i: (0, 0)),
        ],
        out_shape=[
            jax.ShapeDtypeStruct((N, D_IN), jnp.float32),
            jax.ShapeDtypeStruct((1, 1), jnp.float32),
            jax.ShapeDtypeStruct((1, 1), jnp.int32),
        ],
        scratch_shapes=[
            pltpu.VMEM((1, 1), jnp.float32),
            pltpu.VMEM((1, 1), jnp.float32),
        ],
    )(h, x, maskf, w1, b1, w2, b2)


def kernel(x, edge_index, edge_attr, mask_token, emb_w, emb_b,
           conv_w1, conv_b1, conv_w2, conv_b2, bn_w, bn_b,
           dec_w1, dec_b1, dec_w2, dec_b2):
    mask = jax.random.uniform(jax.random.key(42), (N,)) < MASK_RATIO
    maskf = mask.astype(jnp.float32)[:, None]

    src = edge_index[0]
    dst = edge_index[1]
    order = jnp.argsort(dst, stable=True)
    pk = jnp.concatenate(
        [src[:, None], dst[:, None],
         jnp.zeros((E, 14), jnp.int32)], axis=1)
    pks = pk[order]
    srcs = pks[:, 0]
    dsts = pks[:, 1]
    iot = jnp.arange(E, dtype=jnp.int32)
    is_new = jnp.concatenate(
        [jnp.array([True]), dsts[1:] != dsts[:-1]])
    row_start_g = lax.cummax(jnp.where(is_new, iot, 0))
    pk2 = jnp.concatenate(
        [srcs[:, None], dsts[:, None], row_start_g[:, None],
         jnp.zeros((E, 13), jnp.int32)], axis=1)
    starts = jnp.searchsorted(
        dsts, (jnp.arange(NW) * RW).astype(jnp.int32)).astype(jnp.int32)
    ends = jnp.concatenate([starts[1:], jnp.array([E], jnp.int32)])
    loc = jnp.arange(CAP, dtype=jnp.int32)[None, :]
    pos = starts[:, None] + loc
    valid = pos < ends[:, None]
    posc = jnp.minimum(pos, E - 1)
    g = pk2[posc]
    e_src = g[..., 0]
    e_dst = g[..., 1]
    row_start = g[..., 2]
    lane = loc // LP
    deferred = ((row_start - starts[:, None]) // LP) < lane
    jsrc = (loc * 97) % N
    jdst = (JROW0 + jnp.arange(NW, dtype=jnp.int32))[:, None]
    m1 = valid & ~deferred
    m2 = valid & deferred

    def _grid(sv, dv, m):
        sg = jnp.where(m, sv, jsrc).reshape(NW, LN, LP).transpose(0, 2, 1)
        dg = jnp.where(m, dv, jdst).reshape(NW, LN, LP).transpose(0, 2, 1)
        return sg, dg
    s1, d1 = _grid(e_src, e_dst, m1)
    s2, d2 = _grid(e_src, e_dst, m2)
    srcw = jnp.concatenate([s1, s2], axis=1)
    dstw = jnp.concatenate([d1, d2], axis=1)

    h = _embed(x, maskf, mask_token.reshape(1, D_IN), emb_w,
               emb_b.reshape(1, H))
    for i in range(L):
        agg = _sc_agg(h, srcw, dstw)
        y, s8 = _layer_a(h, agg, conv_w1[i], conv_b1[i].reshape(1, 2 * H),
                         conv_w2[i], conv_b2[i].reshape(1, H))
        st = _layer_s(y, s8)
        h = _layer_b(y, st, bn_w[i].reshape(1, H), bn_b[i].reshape(1, H))

    recon, loss11, cnt11 = _dec(h, x, maskf, dec_w1, dec_b1.reshape(1, H),
                                dec_w2, dec_b2.reshape(1, D_IN))
    return (loss11.reshape(()), recon, mask, x, cnt11.reshape(()))
